# baseline (speedup 1.0000x reference)
import jax
import jax.numpy as jnp
from jax.experimental import pallas as pl
from jax.experimental.pallas import tpu as pltpu

_EPS = 1e-5
_VMEM_LIMIT = 48 * 1024 * 1024


def _accum_kernel(x_ref, s_ref, q_ref):
    n = pl.program_id(1)

    @pl.when(n == 0)
    def _():
        s_ref[...] = jnp.zeros_like(s_ref)
        q_ref[...] = jnp.zeros_like(q_ref)

    x = x_ref[...]
    s_ref[...] += x
    q_ref[...] += x * x


def _apply_kernel(x_ref, sm_ref, hm_ref, o_ref):
    z = x_ref[...] * sm_ref[...] + hm_ref[...]
    e = jnp.exp(-jnp.maximum(z, -80.0))
    d = 1.0 + e
    r = pl.reciprocal(d, approx=True)
    r = r * (2.0 - d * r)
    o_ref[...] = z * r


def kernel(x_nchw, gamma, beta):
    N, C, H, W = x_nchw.shape
    HW = H * W
    rows = (C * HW) // 128
    cnt = N * HW
    half = N // 2

    x3 = x_nchw.reshape(N, rows, 128)

    sums, sqs = pl.pallas_call(
        _accum_kernel,
        out_shape=[jax.ShapeDtypeStruct((2, rows, 128), jnp.float32),
                   jax.ShapeDtypeStruct((2, rows, 128), jnp.float32)],
        grid=(2, half),
        in_specs=[pl.BlockSpec((1, rows, 128), lambda h, n: (h * half + n, 0, 0))],
        out_specs=[pl.BlockSpec((1, rows, 128), lambda h, n: (h, 0, 0)),
                   pl.BlockSpec((1, rows, 128), lambda h, n: (h, 0, 0))],
        compiler_params=pltpu.CompilerParams(
            dimension_semantics=("parallel", "arbitrary"),
            vmem_limit_bytes=_VMEM_LIMIT),
    )(x3)

    sum_c = jnp.sum(jnp.sum(sums, axis=0).reshape(C, HW), axis=1)
    ssq_c = jnp.sum(jnp.sum(sqs, axis=0).reshape(C, HW), axis=1)
    mean_c = sum_c / cnt
    var_c = ssq_c / cnt - mean_c * mean_c
    inv_std = jax.lax.rsqrt(var_c + _EPS)
    scale_c = gamma.astype(jnp.float32) * inv_std
    shift_c = beta.astype(jnp.float32) - mean_c * scale_c
    scale_map = jnp.broadcast_to(scale_c[:, None], (C, HW)).reshape(rows, 128)
    shift_map = jnp.broadcast_to(shift_c[:, None], (C, HW)).reshape(rows, 128)

    out3 = pl.pallas_call(
        _apply_kernel,
        out_shape=jax.ShapeDtypeStruct((N, rows, 128), jnp.float32),
        grid=(N,),
        in_specs=[pl.BlockSpec((1, rows, 128), lambda n: (n, 0, 0)),
                  pl.BlockSpec((rows, 128), lambda n: (0, 0)),
                  pl.BlockSpec((rows, 128), lambda n: (0, 0))],
        out_specs=pl.BlockSpec((1, rows, 128), lambda n: (n, 0, 0)),
        compiler_params=pltpu.CompilerParams(
            dimension_semantics=("parallel",),
            vmem_limit_bytes=_VMEM_LIMIT),
    )(x3, scale_map, shift_map)

    return out3.reshape(N, C, H, W)
